# baseline (speedup 1.0000x reference)
import jax
import jax.numpy as jnp
from jax.experimental import pallas as pl

_ROWS_PER_BLOCK = 16


def _log_softmax_block(x_ref, o_ref):
    x = x_ref[...]
    m = jnp.max(x, axis=-1, keepdims=True)
    s = jnp.sum(jnp.exp(x - m), axis=-1, keepdims=True)
    o_ref[...] = (x - m) - jnp.log(s)


def kernel(logits):
    n_rows, vocab = logits.shape
    grid = (n_rows // _ROWS_PER_BLOCK,)
    return pl.pallas_call(
        _log_softmax_block,
        grid=grid,
        in_specs=[pl.BlockSpec((_ROWS_PER_BLOCK, vocab), lambda i: (i, 0))],
        out_specs=pl.BlockSpec((_ROWS_PER_BLOCK, vocab), lambda i: (i, 0)),
        out_shape=jax.ShapeDtypeStruct((n_rows, vocab), logits.dtype),
    )(logits)
